# Initial kernel scaffold; baseline (speedup 1.0000x reference)
import jax
import jax.numpy as jnp
from jax.experimental import pallas as pl


def kernel(x, edge_index, W1, b1, W2, b2, Wb1, bb1, Wb2, bb2):
    raise NotImplementedError("write your pallas kernel here")


# speedup vs baseline: 1.8914x; 1.8914x over previous
import functools

import jax
import jax.numpy as jnp
from jax import lax
from jax.experimental import pallas as pl
from jax.experimental.pallas import tpu as pltpu
from jax.experimental.pallas import tpu_sc as plsc

N_NODES = 10000
N_EDGES = 320000
D_FEAT = 128

_NC = 2
_NS = 16
_NW = _NC * _NS

_CHUNK = 128
_B_TOT = 2 * N_EDGES
_K_CH = 160
_B_PAD = _NW * _K_CH * _CHUNK
_PER_W = _K_CH * _CHUNK

_E_PER_W = N_EDGES // _NW

_NODE_BLK = 1000
_EDGE_BLK = 4000


def _node_body(x_ref, w1_ref, b1_ref, w2_ref, b2_ref, wba_ref, wbb_ref,
               val_ref, pv_ref, xa_ref, xb_ref):
    x = x_ref[...]
    h = jnp.maximum(jnp.dot(x, w1_ref[...],
                            preferred_element_type=jnp.float32) + b1_ref[...],
                    0.0)
    logits = jnp.dot(h, w2_ref[...],
                     preferred_element_type=jnp.float32) + b2_ref[...]
    m = jnp.max(logits, axis=-1, keepdims=True)
    e = jnp.exp(logits - m)
    val_ref[...] = e / jnp.sum(e, axis=-1, keepdims=True)
    idx8 = lax.broadcasted_iota(jnp.int32, logits.shape, 1)
    big = jnp.where(logits == m, idx8, logits.shape[-1])
    am = jnp.min(big, axis=-1, keepdims=True)
    pv_ref[...] = am.astype(jnp.float32) + 1.0
    xa_ref[...] = jnp.dot(x, wba_ref[...], preferred_element_type=jnp.float32)
    xb_ref[...] = jnp.dot(x, wbb_ref[...], preferred_element_type=jnp.float32)


def _node_stage(x, W1, b1, W2, b2, Wba, Wbb):
    nblk = N_NODES // _NODE_BLK
    full = lambda i: (0, 0)
    return pl.pallas_call(
        _node_body,
        grid=(nblk,),
        in_specs=[
            pl.BlockSpec((_NODE_BLK, D_FEAT), lambda i: (i, 0)),
            pl.BlockSpec((D_FEAT, 32), full),
            pl.BlockSpec((1, 32), full),
            pl.BlockSpec((32, 8), full),
            pl.BlockSpec((1, 8), full),
            pl.BlockSpec((D_FEAT, 32), full),
            pl.BlockSpec((D_FEAT, 32), full),
        ],
        out_specs=[
            pl.BlockSpec((_NODE_BLK, 8), lambda i: (i, 0)),
            pl.BlockSpec((_NODE_BLK, 1), lambda i: (i, 0)),
            pl.BlockSpec((_NODE_BLK, 32), lambda i: (i, 0)),
            pl.BlockSpec((_NODE_BLK, 32), lambda i: (i, 0)),
        ],
        out_shape=[
            jax.ShapeDtypeStruct((N_NODES, 8), jnp.float32),
            jax.ShapeDtypeStruct((N_NODES, 1), jnp.float32),
            jax.ShapeDtypeStruct((N_NODES, 32), jnp.float32),
            jax.ShapeDtypeStruct((N_NODES, 32), jnp.float32),
        ],
    )(x, W1, b1, W2, b2, Wba, Wbb)


@functools.lru_cache(maxsize=None)
def _sc_mesh():
    return plsc.VectorSubcoreMesh(core_axis_name="c", subcore_axis_name="s")


@functools.lru_cache(maxsize=None)
def _build_sc_gather():
    return pl.kernel(
        _sc_gather_body,
        out_type=jax.ShapeDtypeStruct((_B_PAD, 32), jnp.float32),
        mesh=_sc_mesh(),
        scratch_types=[
            pltpu.VMEM((_K_CH, _CHUNK), jnp.int32),
            pltpu.VMEM((2, _CHUNK, 32), jnp.float32),
            pltpu.SemaphoreType.DMA,
        ],
        compiler_params=pltpu.CompilerParams(use_tc_tiling_on_sc=False),
    )


def _sc_gather_body(table_hbm, idx_hbm, out_hbm, idx_v, rows_v, gsem):
    wid = lax.axis_index("s") * _NC + lax.axis_index("c")
    base = wid * _PER_W
    pltpu.sync_copy(idx_hbm.at[pl.ds(wid * _K_CH, _K_CH)], idx_v)

    pltpu.async_copy(table_hbm.at[idx_v.at[0]], rows_v.at[0], gsem)

    def body(j, _):
        slot = lax.rem(j, 2)

        @pl.when(j + 1 < _K_CH)
        def _():
            pltpu.async_copy(table_hbm.at[idx_v.at[j + 1]],
                             rows_v.at[lax.rem(j + 1, 2)], gsem)

        pltpu.make_async_copy(table_hbm.at[idx_v.at[j]],
                              rows_v.at[slot], gsem).wait()
        start = pl.multiple_of(base + j * _CHUNK, _CHUNK)
        pltpu.sync_copy(rows_v.at[slot], out_hbm.at[pl.ds(start, _CHUNK)])
        return 0

    lax.fori_loop(0, _K_CH, body, 0)


def _edge_body(ga_ref, gb_ref, bb1_ref, wb2_ref, bb2_ref, bt_ref, ord_ref):
    hb = jnp.maximum(ga_ref[...] + gb_ref[...] + bb1_ref[...], 0.0)
    logits = jnp.dot(hb, wb2_ref[...],
                     preferred_element_type=jnp.float32) + bb2_ref[...]
    m = jnp.max(logits, axis=-1, keepdims=True)
    e = jnp.exp(logits - m)
    bt = e / jnp.sum(e, axis=-1, keepdims=True)
    bt_ref[...] = bt
    ord_ref[...] = (bt[:, 0:1] + 2.0 * bt[:, 1:2] + 3.0 * bt[:, 2:3]
                    + 1.5 * bt[:, 3:4])


def _edge_stage(gout, bb1, Wb2, bb2):
    nblk = N_EDGES // _EDGE_BLK
    full = lambda i: (0, 0)
    return pl.pallas_call(
        _edge_body,
        grid=(nblk,),
        in_specs=[
            pl.BlockSpec((_EDGE_BLK, 32), lambda i: (i, 0)),
            pl.BlockSpec((_EDGE_BLK, 32), lambda i: (i + nblk, 0)),
            pl.BlockSpec((1, 32), full),
            pl.BlockSpec((32, 4), full),
            pl.BlockSpec((1, 4), full),
        ],
        out_specs=[
            pl.BlockSpec((_EDGE_BLK, 4), lambda i: (i, 0)),
            pl.BlockSpec((_EDGE_BLK, 1), lambda i: (i, 0)),
        ],
        out_shape=[
            jax.ShapeDtypeStruct((N_EDGES, 4), jnp.float32),
            jax.ShapeDtypeStruct((N_EDGES, 1), jnp.float32),
        ],
    )(gout, gout, bb1, Wb2, bb2)


@functools.lru_cache(maxsize=None)
def _build_sc_scatter():
    return pl.kernel(
        _sc_scatter_body,
        out_type=jax.ShapeDtypeStruct((_NW * N_NODES,), jnp.float32),
        mesh=_sc_mesh(),
        scratch_types=[
            pltpu.VMEM((_E_PER_W,), jnp.int32),
            pltpu.VMEM((_E_PER_W,), jnp.float32),
            pltpu.VMEM((N_NODES,), jnp.float32),
        ],
        compiler_params=pltpu.CompilerParams(use_tc_tiling_on_sc=False,
                                             needs_layout_passes=False),
    )


def _sc_scatter_body(row_hbm, ord_hbm, out_hbm, idx_v, val_v, acc_v):
    wid = lax.axis_index("s") * _NC + lax.axis_index("c")
    base = wid * _E_PER_W
    pltpu.sync_copy(row_hbm.at[pl.ds(base, _E_PER_W)], idx_v)
    pltpu.sync_copy(ord_hbm.at[pl.ds(base, _E_PER_W)], val_v)

    zero = jnp.zeros((16,), jnp.float32)

    def zbody(i, _):
        acc_v[pl.ds(pl.multiple_of(i * 16, 16), 16)] = zero
        return 0

    lax.fori_loop(0, N_NODES // 16, zbody, 0)

    def body(i, _):
        off = pl.ds(pl.multiple_of(i * 16, 16), 16)
        plsc.addupdate_scatter(acc_v, [idx_v[off]], val_v[off])
        return 0

    lax.fori_loop(0, _E_PER_W // 16, body, 0)
    pltpu.sync_copy(
        acc_v,
        out_hbm.at[pl.ds(pl.multiple_of(wid * N_NODES, 16), N_NODES)])


def _final_body(part_ref, pv_ref, out_ref):
    deg = jnp.sum(part_ref[...], axis=0, keepdims=True)
    d = deg - pv_ref[...]
    out_ref[...] = jnp.sum(d * d, axis=-1, keepdims=True) / N_NODES


def _final_stage(partials, pv_row):
    return pl.pallas_call(
        _final_body,
        out_shape=jax.ShapeDtypeStruct((1, 1), jnp.float32),
    )(partials, pv_row)


def kernel(x, edge_index, W1, b1, W2, b2, Wb1, bb1, Wb2, bb2):
    row = edge_index[0].astype(jnp.int32)
    col = edge_index[1].astype(jnp.int32)

    valences, pv, xa, xb = _node_stage(
        x, W1, b1.reshape(1, 32), W2, b2.reshape(1, 8),
        Wb1[:D_FEAT], Wb1[D_FEAT:])

    table = jnp.concatenate([xa, xb], axis=0)
    idx_all = jnp.concatenate([row, col + N_NODES])
    idx_pad = jnp.pad(idx_all, (0, _B_PAD - _B_TOT)).reshape(_NW * _K_CH,
                                                             _CHUNK)
    gout = _build_sc_gather()(table, idx_pad)

    bond_types, order = _edge_stage(gout, bb1.reshape(1, 32), Wb2,
                                    bb2.reshape(1, 4))

    partials = _build_sc_scatter()(row, order.reshape(N_EDGES))
    partials = partials.reshape(_NW, N_NODES)

    vv = _final_stage(partials, pv.reshape(1, N_NODES))
    return (vv.reshape(()), valences, bond_types)


# speedup vs baseline: 3.7727x; 1.9947x over previous
import functools

import jax
import jax.numpy as jnp
from jax import lax
from jax.experimental import pallas as pl
from jax.experimental.pallas import tpu as pltpu
from jax.experimental.pallas import tpu_sc as plsc

N_NODES = 10000
N_EDGES = 320000
D_FEAT = 128

_NC = 2
_NS = 16
_NW = _NC * _NS

_E_PER_W = N_EDGES // _NW
_CHUNK = 80
_K_CH = _E_PER_W // _CHUNK

_G_ROWS = N_EDGES // 4

_NODE_BLK = 1000
_EDGE_BLK = 2000


def _node_body(x_ref, w1_ref, b1_ref, w2_ref, b2_ref, wba_ref, wbb_ref,
               val_ref, pv_ref, xa_ref, xb_ref):
    x = x_ref[...]
    h = jnp.maximum(jnp.dot(x, w1_ref[...],
                            preferred_element_type=jnp.float32) + b1_ref[...],
                    0.0)
    logits = jnp.dot(h, w2_ref[...],
                     preferred_element_type=jnp.float32) + b2_ref[...]
    m = jnp.max(logits, axis=-1, keepdims=True)
    e = jnp.exp(logits - m)
    val_ref[...] = e / jnp.sum(e, axis=-1, keepdims=True)
    idx8 = lax.broadcasted_iota(jnp.int32, logits.shape, 1)
    big = jnp.where(logits == m, idx8, logits.shape[-1])
    am = jnp.min(big, axis=-1, keepdims=True)
    pv_ref[...] = am.astype(jnp.float32) + 1.0
    xa_ref[...] = jnp.dot(x, wba_ref[...], preferred_element_type=jnp.float32)
    xb_ref[...] = jnp.dot(x, wbb_ref[...], preferred_element_type=jnp.float32)


def _node_stage(x, W1, b1, W2, b2, Wba, Wbb):
    nblk = N_NODES // _NODE_BLK
    full = lambda i: (0, 0)
    return pl.pallas_call(
        _node_body,
        grid=(nblk,),
        in_specs=[
            pl.BlockSpec((_NODE_BLK, D_FEAT), lambda i: (i, 0)),
            pl.BlockSpec((D_FEAT, 32), full),
            pl.BlockSpec((1, 32), full),
            pl.BlockSpec((32, 8), full),
            pl.BlockSpec((1, 8), full),
            pl.BlockSpec((D_FEAT, 32), full),
            pl.BlockSpec((D_FEAT, 32), full),
        ],
        out_specs=[
            pl.BlockSpec((_NODE_BLK, 8), lambda i: (i, 0)),
            pl.BlockSpec((_NODE_BLK, 1), lambda i: (i, 0)),
            pl.BlockSpec((_NODE_BLK, 32), lambda i: (i, 0)),
            pl.BlockSpec((_NODE_BLK, 32), lambda i: (i, 0)),
        ],
        out_shape=[
            jax.ShapeDtypeStruct((N_NODES, 8), jnp.float32),
            jax.ShapeDtypeStruct((N_NODES, 1), jnp.float32),
            jax.ShapeDtypeStruct((N_NODES, 32), jnp.float32),
            jax.ShapeDtypeStruct((N_NODES, 32), jnp.float32),
        ],
    )(x, W1, b1, W2, b2, Wba, Wbb)


@functools.lru_cache(maxsize=None)
def _sc_mesh():
    return plsc.VectorSubcoreMesh(core_axis_name="c", subcore_axis_name="s")


@functools.lru_cache(maxsize=None)
def _build_sc_gather():
    return pl.kernel(
        _sc_gather_body,
        out_type=jax.ShapeDtypeStruct((2 * N_EDGES, 32), jnp.float32),
        mesh=_sc_mesh(),
        scratch_types=[
            pltpu.VMEM((_E_PER_W,), jnp.int32),
            pltpu.VMEM((2, _CHUNK, 32), jnp.float32),
            pltpu.SemaphoreType.DMA,
            pltpu.SemaphoreType.DMA,
        ],
        compiler_params=pltpu.CompilerParams(use_tc_tiling_on_sc=False),
    )


def _sc_gather_body(xa_hbm, xb_hbm, ei_hbm, out_hbm, idx_v, rows_v, gsem,
                    ssem):
    wid = lax.axis_index("s") * _NC + lax.axis_index("c")
    base = wid * _E_PER_W

    def run_side(side, table, out_base):
        pltpu.sync_copy(ei_hbm.at[side, pl.ds(base, _E_PER_W)], idx_v)

        def idx_at(j):
            return idx_v.at[pl.ds(pl.multiple_of(j * _CHUNK, 8), _CHUNK)]

        def out_at(j):
            return out_hbm.at[
                pl.ds(pl.multiple_of(out_base + j * _CHUNK, 8), _CHUNK)]

        pltpu.async_copy(table.at[idx_at(0)], rows_v.at[0], gsem)

        def body(j, _):
            slot = lax.rem(j, 2)

            @pl.when(j + 1 < _K_CH)
            def _():
                @pl.when(j >= 1)
                def _():
                    pltpu.make_async_copy(rows_v.at[lax.rem(j + 1, 2)],
                                          out_at(j - 1), ssem).wait()

                pltpu.async_copy(table.at[idx_at(j + 1)],
                                 rows_v.at[lax.rem(j + 1, 2)], gsem)

            pltpu.make_async_copy(table.at[idx_at(j)], rows_v.at[slot],
                                  gsem).wait()
            pltpu.async_copy(rows_v.at[slot], out_at(j), ssem)
            return 0

        lax.fori_loop(0, _K_CH, body, 0)
        pltpu.make_async_copy(rows_v.at[0], out_at(_K_CH - 1), ssem).wait()
        pltpu.make_async_copy(rows_v.at[0], out_at(_K_CH - 1), ssem).wait()

    run_side(0, xa_hbm, base)
    run_side(1, xb_hbm, N_EDGES + base)


def _edge_body(ga_ref, gb_ref, bb1_ref, wb2_ref, bb2_ref, gsum_ref,
               gcomp_ref, bt_ref, ord_ref):
    hb = jnp.maximum(ga_ref[...] + gb_ref[...] + bb1_ref[...], 0.0)
    logits = jnp.dot(hb, wb2_ref[...],
                     preferred_element_type=jnp.float32) + bb2_ref[...]
    m = jnp.max(logits, axis=-1, keepdims=True)
    e = jnp.exp(logits - m)
    s = jnp.dot(e, gsum_ref[...], preferred_element_type=jnp.float32)
    bt = e / s
    bt_ref[...] = bt
    ord_ref[...] = jnp.dot(bt, gcomp_ref[...],
                           preferred_element_type=jnp.float32)


def _edge_stage(gout, bb1_t4, Wb2bd, bb2_t4, gsum, gcomp):
    nblk = _G_ROWS // _EDGE_BLK
    full = lambda i: (0, 0)
    return pl.pallas_call(
        _edge_body,
        grid=(nblk,),
        in_specs=[
            pl.BlockSpec((_EDGE_BLK, 128), lambda i: (i, 0)),
            pl.BlockSpec((_EDGE_BLK, 128), lambda i: (i + nblk, 0)),
            pl.BlockSpec((1, 128), full),
            pl.BlockSpec((128, 16), full),
            pl.BlockSpec((1, 16), full),
            pl.BlockSpec((16, 16), full),
            pl.BlockSpec((16, 4), full),
        ],
        out_specs=[
            pl.BlockSpec((_EDGE_BLK, 16), lambda i: (i, 0)),
            pl.BlockSpec((_EDGE_BLK, 4), lambda i: (i, 0)),
        ],
        out_shape=[
            jax.ShapeDtypeStruct((_G_ROWS, 16), jnp.float32),
            jax.ShapeDtypeStruct((_G_ROWS, 4), jnp.float32),
        ],
    )(gout, gout, bb1_t4, Wb2bd, bb2_t4, gsum, gcomp)


@functools.lru_cache(maxsize=None)
def _build_sc_scatter():
    return pl.kernel(
        _sc_scatter_body,
        out_type=jax.ShapeDtypeStruct((_NW * N_NODES,), jnp.float32),
        mesh=_sc_mesh(),
        scratch_types=[
            pltpu.VMEM((_E_PER_W,), jnp.int32),
            pltpu.VMEM((_E_PER_W,), jnp.float32),
            pltpu.VMEM((N_NODES,), jnp.float32),
        ],
        compiler_params=pltpu.CompilerParams(use_tc_tiling_on_sc=False,
                                             needs_layout_passes=False),
    )


def _sc_scatter_body(ei_hbm, ord_hbm, out_hbm, idx_v, val_v, acc_v):
    wid = lax.axis_index("s") * _NC + lax.axis_index("c")
    base = wid * _E_PER_W
    pltpu.sync_copy(ei_hbm.at[0, pl.ds(base, _E_PER_W)], idx_v)
    pltpu.sync_copy(ord_hbm.at[pl.ds(base, _E_PER_W)], val_v)

    zero = jnp.zeros((16,), jnp.float32)

    def zbody(i, _):
        acc_v[pl.ds(pl.multiple_of(i * 16, 16), 16)] = zero
        return 0

    lax.fori_loop(0, N_NODES // 16, zbody, 0)

    def body(i, _):
        off = pl.ds(pl.multiple_of(i * 16, 16), 16)
        plsc.addupdate_scatter(acc_v, [idx_v[off]], val_v[off])
        return 0

    lax.fori_loop(0, _E_PER_W // 16, body, 0)
    pltpu.sync_copy(
        acc_v,
        out_hbm.at[pl.ds(pl.multiple_of(wid * N_NODES, 16), N_NODES)])


def _final_body(part_ref, pv_ref, out_ref):
    deg = jnp.sum(part_ref[...], axis=0, keepdims=True)
    d = deg - pv_ref[...]
    out_ref[...] = jnp.sum(d * d, axis=-1, keepdims=True) / N_NODES


def _final_stage(partials, pv_row):
    return pl.pallas_call(
        _final_body,
        out_shape=jax.ShapeDtypeStruct((1, 1), jnp.float32),
    )(partials, pv_row)


def kernel(x, edge_index, W1, b1, W2, b2, Wb1, bb1, Wb2, bb2):
    ei = edge_index.astype(jnp.int32)

    valences, pv, xa, xb = _node_stage(
        x, W1, b1.reshape(1, 32), W2, b2.reshape(1, 8),
        Wb1[:D_FEAT], Wb1[D_FEAT:])

    gout = _build_sc_gather()(xa, xb, ei)
    gout = gout.reshape(2 * _G_ROWS, 128)

    eye4 = jnp.eye(4, dtype=jnp.float32)
    bt4, ord4 = _edge_stage(
        gout,
        jnp.tile(bb1.reshape(1, 32), (1, 4)),
        jnp.kron(eye4, Wb2),
        jnp.tile(bb2.reshape(1, 4), (1, 4)),
        jnp.kron(eye4, jnp.ones((4, 4), jnp.float32)),
        jnp.kron(eye4, jnp.array([[1.0], [2.0], [3.0], [1.5]], jnp.float32)),
    )

    partials = _build_sc_scatter()(ei, ord4.reshape(N_EDGES))
    partials = partials.reshape(_NW, N_NODES)

    vv = _final_stage(partials, pv.reshape(1, N_NODES))
    return (vv.reshape(()), valences, bt4.reshape(N_EDGES, 4))
